# initial kernel scaffold (unmeasured)
import jax
import jax.numpy as jnp
from jax import lax
from jax.experimental import pallas as pl
from jax.experimental.pallas import tpu as pltpu

N_DEV = 4
M = 4096
KS = 1024
N = 8192
NT = 256
GRID = N // NT
KSL = 256


def kernel(x, w_mat, scale_x, scale_w):
    def body(x_hbm, w_hbm, sx_ref, sw_ref, out_ref,
             xg, wg, xseed_sem, wseed_sems, send_x, recv_x, send_w, recv_w):
        j = pl.program_id(0)
        my = lax.axis_index("i")
        left = lax.rem(my + (N_DEV - 1), N_DEV)
        right = lax.rem(my + 1, N_DEV)

        @pl.when(j == 0)
        def _comm():
            cx = pltpu.make_async_copy(x_hbm, xg.at[my], xseed_sem.at[0])
            cx.start()
            wcopies = []
            for t in range(GRID):
                c = pltpu.make_async_copy(
                    w_hbm.at[:, pl.ds(t * NT, NT)], wg.at[my, t],
                    wseed_sems.at[t])
                c.start()
                wcopies.append(c)

            barrier_sem = pltpu.get_barrier_semaphore()
            for nbr in (left, right):
                pl.semaphore_signal(barrier_sem, inc=1, device_id=(nbr,),
                                    device_id_type=pl.DeviceIdType.MESH)
            pl.semaphore_wait(barrier_sem, 2)
            cx.wait()
            for c in wcopies:
                c.wait()

            for h in range(N_DEV - 1):
                o = lax.rem(my - h + N_DEV, N_DEV)
                rx = pltpu.make_async_remote_copy(
                    src_ref=xg.at[o], dst_ref=xg.at[o],
                    send_sem=send_x.at[h], recv_sem=recv_x.at[h],
                    device_id=(right,),
                    device_id_type=pl.DeviceIdType.MESH)
                rw = pltpu.make_async_remote_copy(
                    src_ref=wg.at[o], dst_ref=wg.at[o],
                    send_sem=send_w.at[h], recv_sem=recv_w.at[h],
                    device_id=(right,),
                    device_id_type=pl.DeviceIdType.MESH)
                rx.start()
                rw.start()
                rx.wait()
                rw.wait()

        scale = sx_ref[0] * sw_ref[0]
        acc = jnp.zeros((M, NT), jnp.float32)
        for o in range(N_DEV):
            for ks in range(0, KS, KSL):
                xo = xg[o, :, ks:ks + KSL].astype(jnp.bfloat16)
                wo = wg[o, j, ks:ks + KSL, :].astype(jnp.bfloat16)
                acc = acc + jnp.dot(xo, wo, preferred_element_type=jnp.float32)
        out_ref[...] = acc * scale

    return pl.pallas_call(
        body,
        grid=(GRID,),
        in_specs=[
            pl.BlockSpec(memory_space=pl.ANY),
            pl.BlockSpec(memory_space=pl.ANY),
            pl.BlockSpec(memory_space=pltpu.MemorySpace.SMEM),
            pl.BlockSpec(memory_space=pltpu.MemorySpace.SMEM),
        ],
        out_specs=pl.BlockSpec((M, NT), lambda j: (0, j)),
        out_shape=jax.ShapeDtypeStruct((M, N), jnp.float32),
        scratch_shapes=[
            pltpu.MemorySpace.VMEM((N_DEV, M, KS), jnp.int8),
            pltpu.MemorySpace.VMEM((N_DEV, GRID, KS, NT), jnp.int8),
            pltpu.SemaphoreType.DMA((1,)),
            pltpu.SemaphoreType.DMA((GRID,)),
            pltpu.SemaphoreType.DMA((N_DEV - 1,)),
            pltpu.SemaphoreType.DMA((N_DEV - 1,)),
            pltpu.SemaphoreType.DMA((N_DEV - 1,)),
            pltpu.SemaphoreType.DMA((N_DEV - 1,)),
        ],
        compiler_params=pltpu.CompilerParams(
            dimension_semantics=("arbitrary",),
            collective_id=0,
            vmem_limit_bytes=64 * 1024 * 1024,
        ),
    )(x, w_mat, scale_x, scale_w)


# baseline (device time: 1058759 ns/iter reference)
import jax
import jax.numpy as jnp
from jax import lax
from jax.experimental import pallas as pl
from jax.experimental.pallas import tpu as pltpu

N_DEV = 4
M = 4096
KS = 1024
N = 8192
NT = 256
GRID = N // NT
KSL = 256


def kernel(x, w_mat, scale_x, scale_w):
    def body(x_hbm, w_hbm, sx_ref, sw_ref, out_ref, wg,
             xg, wv, xseed_sem, wseed_sem, wv_sems,
             send_x, recv_x, send_w, recv_w):
        j = pl.program_id(0)
        my = lax.axis_index("i")
        left = lax.rem(my + (N_DEV - 1), N_DEV)
        right = lax.rem(my + 1, N_DEV)

        def wv_copy(t, slot):
            return pltpu.make_async_copy(
                wg.at[:, :, pl.ds(t * NT, NT)], wv.at[slot], wv_sems.at[slot])

        @pl.when(j == 0)
        def _comm():
            cx = pltpu.make_async_copy(x_hbm, xg.at[my], xseed_sem.at[0])
            cw = pltpu.make_async_copy(w_hbm, wg.at[my], wseed_sem.at[0])
            cx.start()
            cw.start()

            barrier_sem = pltpu.get_barrier_semaphore()
            for nbr in (left, right):
                pl.semaphore_signal(barrier_sem, inc=1, device_id=(nbr,),
                                    device_id_type=pl.DeviceIdType.MESH)
            pl.semaphore_wait(barrier_sem, 2)
            cx.wait()
            cw.wait()

            for h in range(N_DEV - 1):
                o = lax.rem(my - h + N_DEV, N_DEV)
                rx = pltpu.make_async_remote_copy(
                    src_ref=xg.at[o], dst_ref=xg.at[o],
                    send_sem=send_x.at[h], recv_sem=recv_x.at[h],
                    device_id=(right,),
                    device_id_type=pl.DeviceIdType.MESH)
                rw = pltpu.make_async_remote_copy(
                    src_ref=wg.at[o], dst_ref=wg.at[o],
                    send_sem=send_w.at[h], recv_sem=recv_w.at[h],
                    device_id=(right,),
                    device_id_type=pl.DeviceIdType.MESH)
                rx.start()
                rw.start()
                rx.wait()
                rw.wait()

            wv_copy(0, 0).start()

        slot = lax.rem(j, 2)
        nslot = lax.rem(j + 1, 2)

        @pl.when(j + 1 < GRID)
        def _prefetch():
            wv_copy(j + 1, nslot).start()

        wv_copy(j, slot).wait()

        scale = sx_ref[0] * sw_ref[0]
        acc = jnp.zeros((M, NT), jnp.float32)
        for o in range(N_DEV):
            for ks in range(0, KS, KSL):
                xo = xg[o, :, ks:ks + KSL].astype(jnp.bfloat16)
                wo = wv[slot, o, ks:ks + KSL, :].astype(jnp.bfloat16)
                acc = acc + jnp.dot(xo, wo, preferred_element_type=jnp.float32)
        out_ref[...] = acc * scale

    return pl.pallas_call(
        body,
        grid=(GRID,),
        in_specs=[
            pl.BlockSpec(memory_space=pl.ANY),
            pl.BlockSpec(memory_space=pl.ANY),
            pl.BlockSpec(memory_space=pltpu.MemorySpace.SMEM),
            pl.BlockSpec(memory_space=pltpu.MemorySpace.SMEM),
        ],
        out_specs=[
            pl.BlockSpec((M, NT), lambda j: (0, j)),
            pl.BlockSpec(memory_space=pl.ANY),
        ],
        out_shape=[
            jax.ShapeDtypeStruct((M, N), jnp.float32),
            jax.ShapeDtypeStruct((N_DEV, KS, N), jnp.int8),
        ],
        scratch_shapes=[
            pltpu.MemorySpace.VMEM((N_DEV, M, KS), jnp.int8),
            pltpu.MemorySpace.VMEM((2, N_DEV, KS, NT), jnp.int8),
            pltpu.SemaphoreType.DMA((1,)),
            pltpu.SemaphoreType.DMA((1,)),
            pltpu.SemaphoreType.DMA((2,)),
            pltpu.SemaphoreType.DMA((N_DEV - 1,)),
            pltpu.SemaphoreType.DMA((N_DEV - 1,)),
            pltpu.SemaphoreType.DMA((N_DEV - 1,)),
            pltpu.SemaphoreType.DMA((N_DEV - 1,)),
        ],
        compiler_params=pltpu.CompilerParams(
            dimension_semantics=("arbitrary",),
            collective_id=0,
            vmem_limit_bytes=64 * 1024 * 1024,
        ),
    )(x, w_mat, scale_x, scale_w)[0]


# device time: 941907 ns/iter; 1.1241x vs baseline; 1.1241x over previous
import jax
import jax.numpy as jnp
from jax import lax
from jax.experimental import pallas as pl
from jax.experimental.pallas import tpu as pltpu

N_DEV = 4
M = 4096
KS = 1024
N = 8192
NT = 256
GRID = N // NT
KSL = 256
MH = M // 2
KH = KS // 2


def kernel(x, w_mat, scale_x, scale_w):
    xb = x.astype(jnp.bfloat16)

    def body(x_hbm, w_hbm, sx_ref, sw_ref, out_ref, wg,
             xgb, wv, xseed_sem, wseed_sem, wv_sems,
             sxr, rxr, sxl, rxl, swr, rwr, swl, rwl):
        j = pl.program_id(0)
        i = pl.program_id(1)
        my = lax.axis_index("i")
        left = lax.rem(my + (N_DEV - 1), N_DEV)
        right = lax.rem(my + 1, N_DEV)

        def wv_copy(t, slot):
            return pltpu.make_async_copy(
                wg.at[:, :, pl.ds(t * NT, NT)], wv.at[slot], wv_sems.at[slot])

        @pl.when((j == 0) & (i == 0))
        def _comm():
            cx0 = pltpu.make_async_copy(
                x_hbm.at[0:MH, :], xgb.at[my, 0], xseed_sem.at[0])
            cx1 = pltpu.make_async_copy(
                x_hbm.at[MH:M, :], xgb.at[my, 1], xseed_sem.at[0])
            cw = pltpu.make_async_copy(w_hbm, wg.at[my], wseed_sem.at[0])
            cx0.start()
            cx1.start()
            cw.start()

            barrier_sem = pltpu.get_barrier_semaphore()
            for nbr in (left, right):
                pl.semaphore_signal(barrier_sem, inc=1, device_id=(nbr,),
                                    device_id_type=pl.DeviceIdType.MESH)
            pl.semaphore_wait(barrier_sem, 2)
            cx0.wait()
            cx1.wait()
            cw.wait()

            for h in range(N_DEV - 1):
                o_r = lax.rem(my - h + N_DEV, N_DEV)
                o_l = lax.rem(my + h, N_DEV)
                rdmas = [
                    pltpu.make_async_remote_copy(
                        src_ref=xgb.at[o_r, 0], dst_ref=xgb.at[o_r, 0],
                        send_sem=sxr.at[h], recv_sem=rxr.at[h],
                        device_id=(right,),
                        device_id_type=pl.DeviceIdType.MESH),
                    pltpu.make_async_remote_copy(
                        src_ref=xgb.at[o_l, 1], dst_ref=xgb.at[o_l, 1],
                        send_sem=sxl.at[h], recv_sem=rxl.at[h],
                        device_id=(left,),
                        device_id_type=pl.DeviceIdType.MESH),
                    pltpu.make_async_remote_copy(
                        src_ref=wg.at[o_r, 0:KH, :], dst_ref=wg.at[o_r, 0:KH, :],
                        send_sem=swr.at[h], recv_sem=rwr.at[h],
                        device_id=(right,),
                        device_id_type=pl.DeviceIdType.MESH),
                    pltpu.make_async_remote_copy(
                        src_ref=wg.at[o_l, KH:KS, :], dst_ref=wg.at[o_l, KH:KS, :],
                        send_sem=swl.at[h], recv_sem=rwl.at[h],
                        device_id=(left,),
                        device_id_type=pl.DeviceIdType.MESH),
                ]
                for r in rdmas:
                    r.start()
                for r in rdmas:
                    r.wait()

            wv_copy(0, 0).start()

        slot = lax.rem(j, 2)
        nslot = lax.rem(j + 1, 2)

        @pl.when(i == 0)
        def _stream():
            wv_copy(j, slot).wait()

            @pl.when(j + 1 < GRID)
            def _prefetch():
                wv_copy(j + 1, nslot).start()

        scale = sx_ref[0] * sw_ref[0]
        for o in range(N_DEV):
            for ks in range(0, KS, KSL):
                xo = xgb[o, i, :, ks:ks + KSL]
                wo = wv[slot, o, ks:ks + KSL, :].astype(jnp.bfloat16)
                d = jnp.dot(xo, wo, preferred_element_type=jnp.float32)
                if o == 0 and ks == 0:
                    out_ref[...] = d
                else:
                    out_ref[...] += d
        out_ref[...] *= scale

    return pl.pallas_call(
        body,
        grid=(GRID, 2),
        in_specs=[
            pl.BlockSpec(memory_space=pl.ANY),
            pl.BlockSpec(memory_space=pl.ANY),
            pl.BlockSpec(memory_space=pltpu.MemorySpace.SMEM),
            pl.BlockSpec(memory_space=pltpu.MemorySpace.SMEM),
        ],
        out_specs=[
            pl.BlockSpec((MH, NT), lambda j, i: (i, j)),
            pl.BlockSpec(memory_space=pl.ANY),
        ],
        out_shape=[
            jax.ShapeDtypeStruct((M, N), jnp.float32),
            jax.ShapeDtypeStruct((N_DEV, KS, N), jnp.int8),
        ],
        scratch_shapes=[
            pltpu.MemorySpace.VMEM((N_DEV, 2, MH, KS), jnp.bfloat16),
            pltpu.MemorySpace.VMEM((2, N_DEV, KS, NT), jnp.int8),
            pltpu.SemaphoreType.DMA((1,)),
            pltpu.SemaphoreType.DMA((1,)),
            pltpu.SemaphoreType.DMA((2,)),
            pltpu.SemaphoreType.DMA((N_DEV - 1,)),
            pltpu.SemaphoreType.DMA((N_DEV - 1,)),
            pltpu.SemaphoreType.DMA((N_DEV - 1,)),
            pltpu.SemaphoreType.DMA((N_DEV - 1,)),
            pltpu.SemaphoreType.DMA((N_DEV - 1,)),
            pltpu.SemaphoreType.DMA((N_DEV - 1,)),
            pltpu.SemaphoreType.DMA((N_DEV - 1,)),
            pltpu.SemaphoreType.DMA((N_DEV - 1,)),
        ],
        compiler_params=pltpu.CompilerParams(
            dimension_semantics=("arbitrary", "arbitrary"),
            collective_id=0,
            vmem_limit_bytes=64 * 1024 * 1024,
        ),
    )(xb, w_mat, scale_x, scale_w)[0]
